# baseline (speedup 1.0000x reference)
import jax
import jax.numpy as jnp
from jax.experimental import pallas as pl

N = 10000
D = 128
HID = 32
ROW_BLOCK = 1000


def _fused_kernel(x_ref, wz_ref, bz_ref, wh_ref, bh_ref,
                  fw1_ref, fb1_ref, fw2_ref, fb2_ref, o_ref):
    x = x_ref[...]
    z = jax.nn.sigmoid(
        jnp.dot(x, wz_ref[...], preferred_element_type=jnp.float32) + bz_ref[...])
    ht = jnp.tanh(
        jnp.dot(x, wh_ref[...], preferred_element_type=jnp.float32) + bh_ref[...])
    h = (1.0 - z) * ht
    h1 = jax.nn.relu(
        jnp.dot(h, fw1_ref[...], preferred_element_type=jnp.float32) + fb1_ref[...])
    logit = jnp.sum(h1 * fw2_ref[...], axis=1, keepdims=True) + fb2_ref[...]
    o_ref[...] = jax.nn.sigmoid(logit)


def kernel(x, edge_index, wz, bz, wr, br, wh, bh, fw1, fb1, fw2, fb2):
    del edge_index, wr, br
    x = x.astype(jnp.float32)
    wz_eff = (wz[0, 0] + wz[1, 0])[:D].astype(jnp.float32)
    wh_eff = (wh[0, 0] + wh[1, 0])[:D].astype(jnp.float32)
    bz2 = bz.reshape(1, HID).astype(jnp.float32)
    bh2 = bh.reshape(1, HID).astype(jnp.float32)
    fb1_2 = fb1.reshape(1, HID).astype(jnp.float32)
    fw2_t = fw2.reshape(1, HID).astype(jnp.float32)
    fb2_2 = fb2.reshape(1, 1).astype(jnp.float32)

    grid = (N // ROW_BLOCK,)
    full = lambda i: (i * 0, i * 0)
    out = pl.pallas_call(
        _fused_kernel,
        grid=grid,
        in_specs=[
            pl.BlockSpec((ROW_BLOCK, D), lambda i: (i, i * 0)),
            pl.BlockSpec((D, HID), full),
            pl.BlockSpec((1, HID), full),
            pl.BlockSpec((D, HID), full),
            pl.BlockSpec((1, HID), full),
            pl.BlockSpec((HID, HID), full),
            pl.BlockSpec((1, HID), full),
            pl.BlockSpec((1, HID), full),
            pl.BlockSpec((1, 1), full),
        ],
        out_specs=pl.BlockSpec((ROW_BLOCK, 1), lambda i: (i, i * 0)),
        out_shape=jax.ShapeDtypeStruct((N, 1), jnp.float32),
    )(x, wz_eff, bz2, wh_eff, bh2,
      fw1.astype(jnp.float32), fb1_2, fw2_t, fb2_2)
    return out


# speedup vs baseline: 1.2378x; 1.2378x over previous
import jax
import jax.numpy as jnp
from jax.experimental import pallas as pl

N = 10000
D = 128
HID = 32
ROW_BLOCK = 5000


def _fused_kernel(x_ref, wz_ref, bz_ref, wh_ref, bh_ref,
                  fw1_ref, fb1_ref, fw2_ref, fb2_ref, o_ref):
    x = x_ref[...]
    z = jax.nn.sigmoid(
        jnp.dot(x, wz_ref[...], preferred_element_type=jnp.float32) + bz_ref[...])
    ht = jnp.tanh(
        jnp.dot(x, wh_ref[...], preferred_element_type=jnp.float32) + bh_ref[...])
    h = (1.0 - z) * ht
    h1 = jax.nn.relu(
        jnp.dot(h, fw1_ref[...], preferred_element_type=jnp.float32) + fb1_ref[...])
    logit = jnp.sum(h1 * fw2_ref[...], axis=1, keepdims=True) + fb2_ref[...]
    o_ref[...] = jax.nn.sigmoid(logit)


def kernel(x, edge_index, wz, bz, wr, br, wh, bh, fw1, fb1, fw2, fb2):
    del edge_index, wr, br
    x = x.astype(jnp.float32)
    wz_eff = (wz[0, 0] + wz[1, 0])[:D].astype(jnp.float32)
    wh_eff = (wh[0, 0] + wh[1, 0])[:D].astype(jnp.float32)
    bz2 = bz.reshape(1, HID).astype(jnp.float32)
    bh2 = bh.reshape(1, HID).astype(jnp.float32)
    fb1_2 = fb1.reshape(1, HID).astype(jnp.float32)
    fw2_t = fw2.reshape(1, HID).astype(jnp.float32)
    fb2_2 = fb2.reshape(1, 1).astype(jnp.float32)

    grid = (N // ROW_BLOCK,)
    full = lambda i: (i * 0, i * 0)
    out = pl.pallas_call(
        _fused_kernel,
        grid=grid,
        in_specs=[
            pl.BlockSpec((ROW_BLOCK, D), lambda i: (i, i * 0)),
            pl.BlockSpec((D, HID), full),
            pl.BlockSpec((1, HID), full),
            pl.BlockSpec((D, HID), full),
            pl.BlockSpec((1, HID), full),
            pl.BlockSpec((HID, HID), full),
            pl.BlockSpec((1, HID), full),
            pl.BlockSpec((1, HID), full),
            pl.BlockSpec((1, 1), full),
        ],
        out_specs=pl.BlockSpec((ROW_BLOCK, 1), lambda i: (i, i * 0)),
        out_shape=jax.ShapeDtypeStruct((N, 1), jnp.float32),
    )(x, wz_eff, bz2, wh_eff, bh2,
      fw1.astype(jnp.float32), fb1_2, fw2_t, fb2_2)
    return out
